# baseline (speedup 1.0000x reference)
import functools

import jax
import jax.numpy as jnp
import numpy as np
from jax.experimental import pallas as pl
from jax.experimental.pallas import tpu as pltpu

NUM_CLASSES = 80
SCALE_RANGES = [(1.0, 96.0), (48.0, 512.0)]
SIGMA = 0.2
ALPHA = 0.25


def _moments_bg_body(mask_ref, c0_ref, c1_ref, mom_ref, bg_ref):
    m = mask_ref[...]
    ys = jax.lax.broadcasted_iota(jnp.int32, (1, 256, 256), 1).astype(jnp.float32)
    xs = jax.lax.broadcasted_iota(jnp.int32, (1, 256, 256), 2).astype(jnp.float32)
    tot = jnp.sum(m, axis=(1, 2))
    sy = jnp.sum(m * ys, axis=(1, 2))
    sx = jnp.sum(m * xs, axis=(1, 2))
    lane = jax.lax.broadcasted_iota(jnp.int32, (48, 128), 1)
    mom_ref[...] = (tot[:, None] * (lane == 0) + sy[:, None] * (lane == 1)
                    + sx[:, None] * (lane == 2))

    def bg_sum(x):
        p = jnp.clip(jax.nn.sigmoid(x), 1e-6, 1.0 - 1e-6)
        return jnp.sum((1.0 - ALPHA) * p * p * (-jnp.log(1.0 - p)))

    tot_bg = bg_sum(c0_ref[...]) + bg_sum(c1_ref[...])
    lane1 = jax.lax.broadcasted_iota(jnp.int32, (1, 128), 1)
    bg_ref[...] = jnp.where(lane1 == 0, tot_bg, 0.0)


def _dice_body(bi_s, ci_s, k_s, row_s, col_s, cls_s, win_s,
               mp_ref, dm_ref, cate_ref, out_ref):
    g = pl.program_id(0)

    @pl.when(g == 0)
    def _():
        out_ref[...] = jnp.zeros_like(out_ref)

    x = mp_ref[0, 0]
    p = jax.nn.sigmoid(x)
    t = dm_ref[0]
    a = jnp.sum(p * t)
    b2 = jnp.sum(p * p) + 1e-4
    c2 = jnp.sum(t * t) + 1e-4
    dice = 1.0 - 2.0 * a / (b2 + c2)
    w = (win_s[g] > 0).astype(jnp.float32)

    img = cate_ref[0, 0]
    ri = jax.lax.broadcasted_iota(jnp.int32, img.shape, 0)
    ci2 = jax.lax.broadcasted_iota(jnp.int32, img.shape, 1)
    x1 = jnp.sum(jnp.where((ri == row_s[g]) & (ci2 == col_s[g]), img, 0.0))
    p1 = jnp.clip(jax.nn.sigmoid(x1), 1e-6, 1.0 - 1e-6)
    corr = (ALPHA * (1.0 - p1) * (1.0 - p1) * (-jnp.log(p1))
            - (1.0 - ALPHA) * p1 * p1 * (-jnp.log(1.0 - p1)))

    lane_o = jax.lax.broadcasted_iota(jnp.int32, out_ref.shape, 1)
    out_ref[...] = (out_ref[...] + (w * dice) * (lane_o == 0)
                    + (w * corr) * (lane_o == 1))


def _scale_indices(target, chs, cws, ng, lo, hi):
    garea = jnp.sqrt(target[:, 4] * target[:, 5])
    half_ws = 0.5 * target[:, 4] * SIGMA
    half_hs = 0.5 * target[:, 5] * SIGMA
    j = (garea >= lo) & (garea <= hi)
    coord_ws = (cws / 256 * ng).astype(jnp.int32)
    coord_hs = (chs / 256 * ng).astype(jnp.int32)
    top_box = jnp.clip(((chs - half_hs) / 256 * ng).astype(jnp.int32), 0, None)
    down_box = jnp.clip(((chs + half_hs) / 256 * ng).astype(jnp.int32), None, ng - 1)
    left_box = jnp.clip(((cws - half_ws) / 256 * ng).astype(jnp.int32), 0, None)
    right_box = jnp.clip(((cws + half_ws) / 256 * ng).astype(jnp.int32), None, ng - 1)
    top = jnp.maximum(top_box, coord_hs - 1)
    down = jnp.minimum(down_box, coord_hs + 1)
    left = jnp.maximum(coord_ws - 1, left_box)
    right = jnp.minimum(right_box, coord_ws + 1)
    off_r = jnp.arange(3)[None, :, None]
    off_c = jnp.arange(3)[None, None, :]
    rr = top[:, None, None] + off_r
    cc = left[:, None, None] + off_c
    valid = (j[:, None, None] & (rr <= down[:, None, None])
             & (cc <= right[:, None, None])).reshape(-1)
    cell = (rr * ng + cc).reshape(-1)
    b = target[:, 0].astype(jnp.int32)
    c = target[:, 1].astype(jnp.int32)
    bid = jnp.repeat(b, 9)
    cvals = jnp.repeat(c, 9)
    flat = bid * (ng * ng) + cell
    order = jnp.arange(flat.shape[0])
    later = ((flat[None, :] == flat[:, None]) & valid[None, :]
             & (order[None, :] > order[:, None]))
    winner = valid & ~later.any(1)
    return bid, cell, cvals, winner


def _dice_call(mp, dmask, cate, bid, cell, cvals, winner, kidx, ng, B):
    bi = jnp.clip(bid, 0, B - 1)
    ci = jnp.clip(cell, 0, ng * ng - 1)
    rowi = ci // ng
    coli = ci % ng
    clsix = jnp.clip(cvals - 1, 0, NUM_CLASSES - 1)
    wini = winner.astype(jnp.int32)
    n = bid.shape[0]
    grid_spec = pltpu.PrefetchScalarGridSpec(
        num_scalar_prefetch=7,
        grid=(n,),
        in_specs=[
            pl.BlockSpec((1, 1, 64, 64),
                         lambda g, bi, ci, k, r, c, cl, w: (bi[g], ci[g], 0, 0)),
            pl.BlockSpec((1, 64, 64),
                         lambda g, bi, ci, k, r, c, cl, w: (k[g], 0, 0)),
            pl.BlockSpec((1, 1, ng, ng),
                         lambda g, bi, ci, k, r, c, cl, w: (bi[g], cl[g], 0, 0)),
        ],
        out_specs=pl.BlockSpec((1, 128),
                               lambda g, bi, ci, k, r, c, cl, w: (0, 0)),
    )
    out = pl.pallas_call(
        _dice_body,
        grid_spec=grid_spec,
        out_shape=jax.ShapeDtypeStruct((1, 128), jnp.float32),
    )(bi, ci, kidx, rowi, coli, clsix, wini, mp, dmask, cate)
    return out[0, 0], out[0, 1]


def kernel(mask_preds0, mask_preds1, cate_preds0, cate_preds1, target, mask):
    B = cate_preds0.shape[0]
    mom, bg = pl.pallas_call(
        _moments_bg_body,
        out_shape=[
            jax.ShapeDtypeStruct((48, 128), jnp.float32),
            jax.ShapeDtypeStruct((1, 128), jnp.float32),
        ],
    )(mask, cate_preds0, cate_preds1)
    bg_total = bg[0, 0]
    tot = mom[:, 0] + 1e-6
    chs = mom[:, 1] / tot
    cws = mom[:, 2] / tot

    dmask = mask[:, ::4, ::4]
    kidx = jnp.asarray(np.repeat(np.arange(target.shape[0]), 9), jnp.int32)

    lmask_sum = jnp.float32(0.0)
    corr_total = jnp.float32(0.0)
    num_mask = jnp.int32(0)
    for i, (mp, cate, ng) in enumerate([
            (mask_preds0, cate_preds0, 40), (mask_preds1, cate_preds1, 36)]):
        lo, hi = SCALE_RANGES[i]
        bid, cell, cvals, winner = _scale_indices(target, chs, cws, ng, lo, hi)
        d, corr = _dice_call(mp, dmask, cate, bid, cell, cvals, winner,
                             kidx, ng, B)
        lmask_sum = lmask_sum + d
        corr_total = corr_total + corr
        num_mask = num_mask + winner.sum().astype(jnp.int32)

    lcls = (bg_total + corr_total) / (num_mask + 1)
    lmask = lmask_sum / num_mask.astype(jnp.float32) * 3.0
    loss = lcls + lmask
    return loss, lcls, lmask


# speedup vs baseline: 1.8459x; 1.8459x over previous
import functools

import jax
import jax.numpy as jnp
import numpy as np
from jax import lax
from jax.experimental import pallas as pl
from jax.experimental.pallas import tpu as pltpu
from jax.experimental.pallas import tpu_sc as plsc

NUM_CLASSES = 80
SCALE_RANGES = [(1.0, 96.0), (48.0, 512.0)]
SIGMA = 0.2
ALPHA = 0.25

_NS = 16
_NCAND = 432
_CPW = 27
_GPW = 3


def _moments_bg_body(mask_ref, c0_ref, c1_ref, mom_ref, bg_ref):
    m = mask_ref[...]
    ys = jax.lax.broadcasted_iota(jnp.int32, (1, 256, 256), 1).astype(jnp.float32)
    xs = jax.lax.broadcasted_iota(jnp.int32, (1, 256, 256), 2).astype(jnp.float32)
    tot = jnp.sum(m, axis=(1, 2))
    sy = jnp.sum(m * ys, axis=(1, 2))
    sx = jnp.sum(m * xs, axis=(1, 2))
    lane = jax.lax.broadcasted_iota(jnp.int32, (48, 128), 1)
    mom_ref[...] = (tot[:, None] * (lane == 0) + sy[:, None] * (lane == 1)
                    + sx[:, None] * (lane == 2))

    def bg_sum(x):
        p = jnp.clip(jax.nn.sigmoid(x), 1e-6, 1.0 - 1e-6)
        return jnp.sum((1.0 - ALPHA) * p * p * (-jnp.log(1.0 - p)))

    tot_bg = bg_sum(c0_ref[...]) + bg_sum(c1_ref[...])
    lane1 = jax.lax.broadcasted_iota(jnp.int32, (1, 128), 1)
    bg_ref[...] = jnp.where(lane1 == 0, tot_bg, 0.0)


def _sc_body(mp0, mp1, dm, cf0, cf1, pidx, cidx,
             out_sums, out_cate,
             pidxa_v, pidxb_v, cidx_v, pred_v, dm_v, cate_v,
             abt_v, sem):
    cid = lax.axis_index("c")
    sid = lax.axis_index("s")

    pltpu.sync_copy(pidx.at[cid, sid, pl.ds(0, 16)], pidxa_v)
    pltpu.sync_copy(pidx.at[cid, sid, pl.ds(16, 16)], pidxb_v)
    pltpu.sync_copy(cidx.at[cid, sid], cidx_v)

    pltpu.sync_copy(dm.at[pl.ds(8 * sid, 8)], dm_v)

    @pl.when(cid == 0)
    def _():
        pltpu.async_copy(cf0.at[cidx_v], cate_v, sem).wait()

    @pl.when(cid == 1)
    def _():
        pltpu.async_copy(cf1.at[cidx_v], cate_v, sem).wait()

    for half, count in ((0, 16), (1, _CPW - 16)):
        idx_half = pidxa_v if half == 0 else pidxb_v

        @pl.when(cid == 0)
        def _():
            pltpu.async_copy(mp0.at[idx_half], pred_v, sem).wait()

        @pl.when(cid == 1)
        def _():
            pltpu.async_copy(mp1.at[idx_half], pred_v, sem).wait()

        def cand_body(j, carry):
            c = 16 * half + j
            gl = c // 9

            def chunk(v, acc):
                a, b, t = acc
                x = pred_v[j, pl.ds(v * 16, 16)]
                tm = dm_v[gl, pl.ds(v * 16, 16)]
                p = 1.0 / (1.0 + jnp.exp(-x))
                return (a + p * tm, b + p * p, t + tm * tm)

            z = jnp.zeros((16,), jnp.float32)
            a, b, t = lax.fori_loop(0, 256, chunk, (z, z, z))
            abt_v[c, pl.ds(0, 16)] = a
            abt_v[c, pl.ds(16, 16)] = b
            abt_v[c, pl.ds(32, 16)] = t
            return carry

        lax.fori_loop(0, count, cand_body, 0)

    base = cid * (_NS * 32) + sid * 32
    pltpu.sync_copy(abt_v, out_sums.at[pl.ds(base, 32)])
    pltpu.sync_copy(cate_v, out_cate.at[pl.ds(base, 32)])


def _sc_call(mp0r, mp1r, dmr, cf0r, cf1r, pidx, cidx):
    mesh = plsc.VectorSubcoreMesh(core_axis_name="c", subcore_axis_name="s")
    f = functools.partial(
        pl.kernel,
        mesh=mesh,
        out_type=[
            jax.ShapeDtypeStruct((1024, 128), jnp.float32),
            jax.ShapeDtypeStruct((1024, 128), jnp.float32),
        ],
        scratch_types=[
            pltpu.VMEM((16,), jnp.int32),
            pltpu.VMEM((16,), jnp.int32),
            pltpu.VMEM((32,), jnp.int32),
            pltpu.VMEM((16, 4096), jnp.float32),
            pltpu.VMEM((8, 4096), jnp.float32),
            pltpu.VMEM((32, 128), jnp.float32),
            pltpu.VMEM((32, 128), jnp.float32),
            pltpu.SemaphoreType.DMA,
        ],
    )(_sc_body)
    return f(mp0r, mp1r, dmr, cf0r, cf1r, pidx, cidx)


def _finish_body(sums_ref, cate_ref, cm_ref, w_ref, bg_ref, out_ref):
    s = sums_ref[...]
    a = jnp.sum(s[:, 0:16], axis=1, keepdims=True)
    b = jnp.sum(s[:, 16:32], axis=1, keepdims=True) + 1e-4
    t = jnp.sum(s[:, 32:48], axis=1, keepdims=True) + 1e-4
    dice = 1.0 - 2.0 * a / (b + t)
    rowi = jax.lax.broadcasted_iota(jnp.int32, (1024, 1), 0)
    keep = (rowi % 32) < _CPW
    w = w_ref[...]
    lmask_sum = jnp.sum(jnp.where(keep, w * dice, 0.0))
    nm = jnp.sum(w)

    x1 = jnp.sum(cate_ref[...] * cm_ref[...], axis=1, keepdims=True)
    p1 = jnp.clip(jax.nn.sigmoid(x1), 1e-6, 1.0 - 1e-6)
    corr = (ALPHA * (1.0 - p1) * (1.0 - p1) * (-jnp.log(p1))
            - (1.0 - ALPHA) * p1 * p1 * (-jnp.log(1.0 - p1)))
    corr_sum = jnp.sum(jnp.where(keep, w * corr, 0.0))

    bgv = bg_ref[0, 0]
    lcls = (bgv + corr_sum) / (nm + 1.0)
    lmask = lmask_sum / nm * 3.0
    loss = lcls + lmask
    lane = jax.lax.broadcasted_iota(jnp.int32, (1, 128), 1)
    out_ref[...] = (loss * (lane == 0) + lcls * (lane == 1)
                    + lmask * (lane == 2))


def _scale_indices(target, chs, cws, ng, lo, hi):
    garea = jnp.sqrt(target[:, 4] * target[:, 5])
    half_ws = 0.5 * target[:, 4] * SIGMA
    half_hs = 0.5 * target[:, 5] * SIGMA
    j = (garea >= lo) & (garea <= hi)
    coord_ws = (cws / 256 * ng).astype(jnp.int32)
    coord_hs = (chs / 256 * ng).astype(jnp.int32)
    top_box = jnp.clip(((chs - half_hs) / 256 * ng).astype(jnp.int32), 0, None)
    down_box = jnp.clip(((chs + half_hs) / 256 * ng).astype(jnp.int32), None, ng - 1)
    left_box = jnp.clip(((cws - half_ws) / 256 * ng).astype(jnp.int32), 0, None)
    right_box = jnp.clip(((cws + half_ws) / 256 * ng).astype(jnp.int32), None, ng - 1)
    top = jnp.maximum(top_box, coord_hs - 1)
    down = jnp.minimum(down_box, coord_hs + 1)
    left = jnp.maximum(coord_ws - 1, left_box)
    right = jnp.minimum(right_box, coord_ws + 1)
    off_r = jnp.arange(3)[None, :, None]
    off_c = jnp.arange(3)[None, None, :]
    rr = top[:, None, None] + off_r
    cc = left[:, None, None] + off_c
    valid = (j[:, None, None] & (rr <= down[:, None, None])
             & (cc <= right[:, None, None])).reshape(-1)
    cell = (rr * ng + cc).reshape(-1)
    b = target[:, 0].astype(jnp.int32)
    c = target[:, 1].astype(jnp.int32)
    bid = jnp.repeat(b, 9)
    cvals = jnp.repeat(c, 9)
    flat = bid * (ng * ng) + cell
    order = jnp.arange(flat.shape[0])
    later = ((flat[None, :] == flat[:, None]) & valid[None, :]
             & (order[None, :] > order[:, None]))
    winner = valid & ~later.any(1)
    return bid, cell, cvals, winner


def kernel(mask_preds0, mask_preds1, cate_preds0, cate_preds1, target, mask):
    B = cate_preds0.shape[0]
    mom, bg = pl.pallas_call(
        _moments_bg_body,
        out_shape=[
            jax.ShapeDtypeStruct((48, 128), jnp.float32),
            jax.ShapeDtypeStruct((1, 128), jnp.float32),
        ],
    )(mask, cate_preds0, cate_preds1)
    tot = mom[:, 0] + 1e-6
    chs = mom[:, 1] / tot
    cws = mom[:, 2] / tot

    dmask = mask[:, ::4, ::4]
    dmr = dmask.reshape(48, 4096)
    mp0r = mask_preds0.reshape(-1, 4096)
    mp1r = mask_preds1.reshape(-1, 4096)
    cf0r = cate_preds0.reshape(-1, 128)
    cf1r = cate_preds1.reshape(-1, 128)

    pidx_list, cidx_list, cm_list, w_list = [], [], [], []
    for i, ng in enumerate([40, 36]):
        lo, hi = SCALE_RANGES[i]
        bid, cell, cvals, winner = _scale_indices(target, chs, cws, ng, lo, hi)
        bi = jnp.clip(bid, 0, B - 1)
        ci = jnp.clip(cell, 0, ng * ng - 1)
        rowidx = bi * (ng * ng) + ci
        cls = jnp.clip(cvals - 1, 0, NUM_CLASSES - 1)
        e = ((bi * NUM_CLASSES + cls) * ng + ci // ng) * ng + ci % ng
        pidx_list.append(jnp.pad(rowidx.reshape(_NS, _CPW), ((0, 0), (0, 5))))
        cidx_list.append(jnp.pad((e // 128).reshape(_NS, _CPW), ((0, 0), (0, 5))))
        cm_list.append((e % 128).astype(jnp.int32))
        w_list.append(winner.astype(jnp.float32))

    pidx = jnp.stack(pidx_list).astype(jnp.int32)
    cidx = jnp.stack(cidx_list).astype(jnp.int32)
    lanes = jnp.pad(jnp.stack(cm_list).reshape(2, _NS, _CPW),
                    ((0, 0), (0, 0), (0, 5))).reshape(1024)
    cm = (lanes[:, None] == jnp.arange(128)[None, :]).astype(jnp.float32)
    w = jnp.pad(jnp.stack(w_list).reshape(2, _NS, _CPW),
                ((0, 0), (0, 0), (0, 5))).reshape(1024)[:, None]

    k48 = jnp.arange(48)
    dmp = jnp.zeros((128, 4096), jnp.float32).at[(k48 // 3) * 8 + k48 % 3].set(dmr)
    sums, caterows = _sc_call(mp0r, mp1r, dmp, cf0r, cf1r, pidx, cidx)

    out = pl.pallas_call(
        _finish_body,
        out_shape=jax.ShapeDtypeStruct((1, 128), jnp.float32),
    )(sums, caterows, cm, w, bg)
    return out[0, 0], out[0, 1], out[0, 2]
